# baseline (speedup 1.0000x reference)
import jax
import jax.numpy as jnp
from jax import lax
from jax.experimental import pallas as pl
from jax.experimental.pallas import tpu as pltpu
from jax.experimental.pallas import tpu_sc as plsc

_VOCAB = 100000
_DIM = 64
_CDIM = 16
_COMB = _DIM + _CDIM
_B = 1024
_L = 200
_LT = 20

_NC = 2
_NS = 16
_NW = _NC * _NS

_TXT = _B * _L
_TOP = _B * _LT
_WRD = _TXT + _TOP
_CHUNK = 128
_WRD_PW = _WRD // _NW
_POS_PW = _TXT // _NW
_WRD_CH = _WRD_PW // _CHUNK
_POS_CH = _POS_PW // _CHUNK


def _sc_body(word_hbm, cm0_hbm, cm1_hbm, cm2_hbm, cm3_hbm,
             wrd_hbm, i0_hbm, i1_hbm, i2_hbm, i3_hbm,
             wrd_out, conn_out, pos_out,
             idx_wrd, idx_pos,
             wb0, wb1, p0, p1,
             wg0, wg1, ws0, ws1,
             pg0, pg1, pa0, pa1, pso0, pso1, cso0, cso1):
  w = lax.axis_index("s") * _NC + lax.axis_index("c")
  wrd_base = w * _WRD_PW
  pos_base = w * _POS_PW

  wb = [wb0, wb1]
  pb = [p0, p1]
  wg = [wg0, wg1]
  ws = [ws0, ws1]
  pg = [pg0, pg1]
  pa = [pa0, pa1]
  pso = [pso0, pso1]
  cso = [cso0, cso1]

  pltpu.sync_copy(wrd_hbm.at[w], idx_wrd)
  pltpu.sync_copy(i0_hbm.at[w], idx_pos.at[0])
  pltpu.sync_copy(i1_hbm.at[w], idx_pos.at[1])
  pltpu.sync_copy(i2_hbm.at[w], idx_pos.at[2])
  pltpu.sync_copy(i3_hbm.at[w], idx_pos.at[3])

  pltpu.async_copy(word_hbm.at[idx_wrd.at[0]], wb[0], wg[0])

  def wrd_pair(t, _):
    for s in range(2):
      j = 2 * t + s
      dst = wrd_out.at[pl.ds(wrd_base + j * _CHUNK, _CHUNK)]
      pltpu.make_async_copy(word_hbm.at[idx_wrd.at[j]], wb[s], wg[s]).wait()
      if s == 1:
        pltpu.make_async_copy(
            wb[0], wrd_out.at[pl.ds(wrd_base + j * _CHUNK - _CHUNK, _CHUNK)],
            ws[0]).wait()
      else:
        @pl.when(t > 0)
        def _():
          pltpu.make_async_copy(
              wb[1], wrd_out.at[pl.ds(wrd_base + j * _CHUNK - _CHUNK, _CHUNK)],
              ws[1]).wait()

      @pl.when(j + 1 < _WRD_CH)
      def _():
        pltpu.async_copy(word_hbm.at[idx_wrd.at[j + 1]], wb[1 - s], wg[1 - s])
      pltpu.async_copy(wb[s], dst, ws[s])
    return _

  lax.fori_loop(0, _WRD_CH // 2, wrd_pair, None)
  jlast = _WRD_CH - 1
  pltpu.make_async_copy(word_hbm.at[idx_wrd.at[jlast]], wb[0], wg[0]).wait()
  pltpu.make_async_copy(
      wb[1], wrd_out.at[pl.ds(wrd_base + (jlast - 1) * _CHUNK, _CHUNK)],
      ws[1]).wait()
  pltpu.async_copy(
      wb[0], wrd_out.at[pl.ds(wrd_base + jlast * _CHUNK, _CHUNK)], ws[0])

  pltpu.async_copy(cm0_hbm.at[idx_pos.at[0, 0]], pb[0], pg[0])

  def pos_pair(t, _):
    for s in range(2):
      j = 2 * t + s
      dst = pl.ds(pos_base + j * _CHUNK, _CHUNK)
      pdst = pl.ds(pos_base + j * _CHUNK - _CHUNK, _CHUNK)
      pltpu.make_async_copy(cm0_hbm.at[idx_pos.at[0, j]], pb[s], pg[s]).wait()
      pltpu.async_copy(cm1_hbm.at[idx_pos.at[1, j]], pb[s], pa[s], add=True)
      pltpu.async_copy(cm2_hbm.at[idx_pos.at[2, j]], pb[s], pa[s], add=True)
      pltpu.async_copy(cm3_hbm.at[idx_pos.at[3, j]], pb[s], pa[s], add=True)
      o = 1 - s
      if s == 1:
        pltpu.make_async_copy(pb[0].at[:, 0:_DIM], pos_out.at[pdst],
                              pso[0]).wait()
        pltpu.make_async_copy(pb[0].at[:, _DIM:_COMB], conn_out.at[pdst],
                              cso[0]).wait()
      else:
        @pl.when(t > 0)
        def _():
          pltpu.make_async_copy(pb[1].at[:, 0:_DIM], pos_out.at[pdst],
                                pso[1]).wait()
          pltpu.make_async_copy(pb[1].at[:, _DIM:_COMB], conn_out.at[pdst],
                                cso[1]).wait()

      @pl.when(j + 1 < _POS_CH)
      def _():
        pltpu.async_copy(cm0_hbm.at[idx_pos.at[0, j + 1]], pb[o], pg[o])
      pltpu.make_async_copy(cm1_hbm.at[idx_pos.at[1, j]], pb[s], pa[s]).wait()
      pltpu.make_async_copy(cm2_hbm.at[idx_pos.at[2, j]], pb[s], pa[s]).wait()
      pltpu.make_async_copy(cm3_hbm.at[idx_pos.at[3, j]], pb[s], pa[s]).wait()
      pltpu.async_copy(pb[s].at[:, 0:_DIM], pos_out.at[dst], pso[s])
      pltpu.async_copy(pb[s].at[:, _DIM:_COMB], conn_out.at[dst], cso[s])
    return _

  lax.fori_loop(0, _POS_CH // 2, pos_pair, None)
  end1 = pl.ds(pos_base + (_POS_CH - 1) * _CHUNK, _CHUNK)
  pltpu.make_async_copy(pb[1].at[:, 0:_DIM], pos_out.at[end1], pso[1]).wait()
  pltpu.make_async_copy(pb[1].at[:, _DIM:_COMB], conn_out.at[end1],
                        cso[1]).wait()
  pltpu.make_async_copy(
      wb[0], wrd_out.at[pl.ds(wrd_base + jlast * _CHUNK, _CHUNK)], ws[0]).wait()


@jax.jit
def _run(word_table, cm0, cm1, cm2, cm3, wrd_idx, i0, i1, i2, i3):
  mesh = plsc.VectorSubcoreMesh(core_axis_name="c", subcore_axis_name="s")
  f = pl.kernel(
      _sc_body,
      out_type=[
          jax.ShapeDtypeStruct((_WRD, _DIM), jnp.float32),
          jax.ShapeDtypeStruct((_TXT, _CDIM), jnp.float32),
          jax.ShapeDtypeStruct((_TXT, _DIM), jnp.float32),
      ],
      mesh=mesh,
      compiler_params=pltpu.CompilerParams(use_tc_tiling_on_sc=False),
      scratch_types=[
          pltpu.VMEM((_WRD_CH, _CHUNK), jnp.int32),
          pltpu.VMEM((4, _POS_CH, _CHUNK), jnp.int32),
          pltpu.VMEM((_CHUNK, _DIM), jnp.float32),
          pltpu.VMEM((_CHUNK, _DIM), jnp.float32),
          pltpu.VMEM((_CHUNK, _COMB), jnp.float32),
          pltpu.VMEM((_CHUNK, _COMB), jnp.float32),
      ] + [pltpu.SemaphoreType.DMA] * 12,
  )
  return f(word_table, cm0, cm1, cm2, cm3, wrd_idx, i0, i1, i2, i3)


def kernel(word_table, conn_NOUN, conn_VERB, conn_ADJ, conn_ADV,
           text, topic, idx_NOUN, idx_VERB, idx_ADJ, idx_ADV,
           txt_l, top_l):
  cm0 = jnp.concatenate([word_table, conn_NOUN], axis=1)
  cm1 = jnp.concatenate([word_table, conn_VERB], axis=1)
  cm2 = jnp.concatenate([word_table, conn_ADJ], axis=1)
  cm3 = jnp.concatenate([word_table, conn_ADV], axis=1)
  wrd_idx = jnp.concatenate([text.reshape(-1), topic.reshape(-1)])
  wrd_idx = wrd_idx.reshape(_NW, _WRD_CH, _CHUNK)
  i0 = idx_NOUN.reshape(_NW, _POS_CH, _CHUNK)
  i1 = idx_VERB.reshape(_NW, _POS_CH, _CHUNK)
  i2 = idx_ADJ.reshape(_NW, _POS_CH, _CHUNK)
  i3 = idx_ADV.reshape(_NW, _POS_CH, _CHUNK)
  wrd_E, conn_E, pos_E = _run(
      word_table, cm0, cm1, cm2, cm3, wrd_idx, i0, i1, i2, i3)
  word_E = wrd_E[:_TXT].reshape(_B, _L, _DIM)
  top_E = wrd_E[_TXT:].reshape(_B, _LT, _DIM)
  return (word_E, top_E,
          conn_E.reshape(_B, _L, _CDIM),
          pos_E.reshape(_B, _L, _DIM),
          txt_l, top_l)


# speedup vs baseline: 1.5009x; 1.5009x over previous
import jax
import jax.numpy as jnp
from jax import lax
from jax.experimental import pallas as pl
from jax.experimental.pallas import tpu as pltpu
from jax.experimental.pallas import tpu_sc as plsc

_VOCAB = 100000
_DIM = 64
_CDIM = 16
_B = 1024
_L = 200
_LT = 20

_NC = 2
_NS = 16
_NW = _NC * _NS

_TXT = _B * _L
_TOP = _B * _LT
_WRD = _TXT + _TOP
_CHUNK = 128
_WRD_PW = _WRD // _NW
_POS_PW = _TXT // _NW
_WRD_CH = _WRD_PW // _CHUNK
_POS_CH = _POS_PW // _CHUNK


def _sc_body(word_hbm, c0_hbm, c1_hbm, c2_hbm, c3_hbm,
             wrd_hbm, i0_hbm, i1_hbm, i2_hbm, i3_hbm,
             wrd_out, conn_out, pos_out,
             idx_wrd, idx_pos,
             wb0, wb1, wb2, wb3, p0, p1, p2, p3, cb0, cb1, cb2, cb3,
             wg0, wg1, wg2, wg3, ws0, ws1, ws2, ws3,
             pg0, pg1, pg2, pg3, pa0, pa1, pa2, pa3,
             pso0, pso1, pso2, pso3):
  w = lax.axis_index("s") * _NC + lax.axis_index("c")
  wrd_base = w * _WRD_PW
  pos_base = w * _POS_PW

  wb = [wb0, wb1, wb2, wb3]
  pb = [p0, p1, p2, p3]
  cb = [cb0, cb1, cb2, cb3]
  wg = [wg0, wg1, wg2, wg3]
  ws = [ws0, ws1, ws2, ws3]
  pg = [pg0, pg1, pg2, pg3]
  pa = [pa0, pa1, pa2, pa3]
  pso = [pso0, pso1, pso2, pso3]

  pltpu.sync_copy(wrd_hbm.at[w], idx_wrd)
  pltpu.sync_copy(i0_hbm.at[w], idx_pos.at[0])
  pltpu.sync_copy(i1_hbm.at[w], idx_pos.at[1])
  pltpu.sync_copy(i2_hbm.at[w], idx_pos.at[2])
  pltpu.sync_copy(i3_hbm.at[w], idx_pos.at[3])

  def j1_fire(j, s):
    pltpu.async_copy(word_hbm.at[idx_wrd.at[j]], wb[s], wg[s])

  def j1_a(j, s):
    pltpu.make_async_copy(word_hbm.at[idx_wrd.at[j]], wb[s], wg[s]).wait()
    pltpu.async_copy(
        wb[s], wrd_out.at[pl.ds(wrd_base + j * _CHUNK, _CHUNK)], ws[s])

  def j1_c(j, s):
    pltpu.make_async_copy(
        wb[s], wrd_out.at[pl.ds(wrd_base + j * _CHUNK, _CHUNK)], ws[s]).wait()

  def j2_fire(j, s):
    pltpu.async_copy(word_hbm.at[idx_pos.at[0, j]], pb[s], pg[s])
    pltpu.async_copy(c0_hbm.at[idx_pos.at[0, j]], cb[s], pg[s])

  def j2_a(j, s):
    pltpu.make_async_copy(word_hbm.at[idx_pos.at[0, j]], pb[s], pg[s]).wait()
    pltpu.make_async_copy(c0_hbm.at[idx_pos.at[0, j]], cb[s], pg[s]).wait()
    pltpu.async_copy(word_hbm.at[idx_pos.at[1, j]], pb[s], pa[s], add=True)
    pltpu.async_copy(word_hbm.at[idx_pos.at[2, j]], pb[s], pa[s], add=True)
    pltpu.async_copy(word_hbm.at[idx_pos.at[3, j]], pb[s], pa[s], add=True)
    pltpu.async_copy(c1_hbm.at[idx_pos.at[1, j]], cb[s], pa[s], add=True)
    pltpu.async_copy(c2_hbm.at[idx_pos.at[2, j]], cb[s], pa[s], add=True)
    pltpu.async_copy(c3_hbm.at[idx_pos.at[3, j]], cb[s], pa[s], add=True)

  def j2_b(j, s):
    dst = pl.ds(pos_base + j * _CHUNK, _CHUNK)
    pltpu.make_async_copy(word_hbm.at[idx_pos.at[1, j]], pb[s], pa[s]).wait()
    pltpu.make_async_copy(word_hbm.at[idx_pos.at[2, j]], pb[s], pa[s]).wait()
    pltpu.make_async_copy(word_hbm.at[idx_pos.at[3, j]], pb[s], pa[s]).wait()
    pltpu.make_async_copy(c1_hbm.at[idx_pos.at[1, j]], cb[s], pa[s]).wait()
    pltpu.make_async_copy(c2_hbm.at[idx_pos.at[2, j]], cb[s], pa[s]).wait()
    pltpu.make_async_copy(c3_hbm.at[idx_pos.at[3, j]], cb[s], pa[s]).wait()
    pltpu.async_copy(pb[s], pos_out.at[dst], pso[s])
    pltpu.async_copy(cb[s], conn_out.at[dst], pso[s])

  def j2_c(j, s):
    dst = pl.ds(pos_base + j * _CHUNK, _CHUNK)
    pltpu.make_async_copy(pb[s], pos_out.at[dst], pso[s]).wait()
    pltpu.make_async_copy(cb[s], conn_out.at[dst], pso[s]).wait()

  j2_fire(0, 0)
  j2_fire(1, 1)

  j1_fire(0, 0)
  j1_fire(1, 1)
  j1_a(0, 0)
  j1_fire(2, 2)
  j1_a(1, 1)
  j1_fire(3, 3)

  def j1_loop(t, _):
    for u in range(4):
      j = 4 * t + 2 + u
      s = (2 + u) % 4
      j1_a(j, s)
      j1_c(j - 2, u)
      @pl.when(j + 2 < _WRD_CH)
      def _():
        j1_fire(j + 2, u)
    return _

  lax.fori_loop(0, (_WRD_CH - 3) // 4, j1_loop, None)
  j1_a(_WRD_CH - 1, (_WRD_CH - 1) % 4)

  j2_a(0, 0)
  j2_fire(2, 2)
  j2_a(1, 1)
  j2_b(0, 0)
  j2_fire(3, 3)

  def j2_loop(t, _):
    for u in range(4):
      j = 4 * t + 2 + u
      s = (2 + u) % 4
      j2_a(j, s)
      j2_b(j - 1, (1 + u) % 4)
      j2_c(j - 2, u)
      @pl.when(j + 2 < _POS_CH)
      def _():
        j2_fire(j + 2, u)
    return _

  lax.fori_loop(0, (_POS_CH - 2) // 4, j2_loop, None)
  j2_b(_POS_CH - 1, (_POS_CH - 1) % 4)
  j2_c(_POS_CH - 2, (_POS_CH - 2) % 4)
  j2_c(_POS_CH - 1, (_POS_CH - 1) % 4)

  j1_c(_WRD_CH - 3, (_WRD_CH - 3) % 4)
  j1_c(_WRD_CH - 2, (_WRD_CH - 2) % 4)
  j1_c(_WRD_CH - 1, (_WRD_CH - 1) % 4)


@jax.jit
def _run(word_table, c0, c1, c2, c3, wrd_idx, i0, i1, i2, i3):
  mesh = plsc.VectorSubcoreMesh(core_axis_name="c", subcore_axis_name="s")
  f = pl.kernel(
      _sc_body,
      out_type=[
          jax.ShapeDtypeStruct((_WRD, _DIM), jnp.float32),
          jax.ShapeDtypeStruct((_TXT, _CDIM), jnp.float32),
          jax.ShapeDtypeStruct((_TXT, _DIM), jnp.float32),
      ],
      mesh=mesh,
      compiler_params=pltpu.CompilerParams(use_tc_tiling_on_sc=False),
      scratch_types=[
          pltpu.VMEM((_WRD_CH, _CHUNK), jnp.int32),
          pltpu.VMEM((4, _POS_CH, _CHUNK), jnp.int32),
      ] + [pltpu.VMEM((_CHUNK, _DIM), jnp.float32)] * 4
        + [pltpu.VMEM((_CHUNK, _DIM), jnp.float32)] * 4
        + [pltpu.VMEM((_CHUNK, _CDIM), jnp.float32)] * 4
        + [pltpu.SemaphoreType.DMA] * 20,
  )
  return f(word_table, c0, c1, c2, c3, wrd_idx, i0, i1, i2, i3)


def kernel(word_table, conn_NOUN, conn_VERB, conn_ADJ, conn_ADV,
           text, topic, idx_NOUN, idx_VERB, idx_ADJ, idx_ADV,
           txt_l, top_l):
  wrd_idx = jnp.concatenate([text.reshape(-1), topic.reshape(-1)])
  wrd_idx = wrd_idx.reshape(_NW, _WRD_CH, _CHUNK)
  i0 = idx_NOUN.reshape(_NW, _POS_CH, _CHUNK)
  i1 = idx_VERB.reshape(_NW, _POS_CH, _CHUNK)
  i2 = idx_ADJ.reshape(_NW, _POS_CH, _CHUNK)
  i3 = idx_ADV.reshape(_NW, _POS_CH, _CHUNK)
  wrd_E, conn_E, pos_E = _run(
      word_table, conn_NOUN, conn_VERB, conn_ADJ, conn_ADV,
      wrd_idx, i0, i1, i2, i3)
  word_E = wrd_E[:_TXT].reshape(_B, _L, _DIM)
  top_E = wrd_E[_TXT:].reshape(_B, _LT, _DIM)
  return (word_E, top_E,
          conn_E.reshape(_B, _L, _CDIM),
          pos_E.reshape(_B, _L, _DIM),
          txt_l, top_l)


# speedup vs baseline: 1.5065x; 1.0038x over previous
import jax
import jax.numpy as jnp
from jax import lax
from jax.experimental import pallas as pl
from jax.experimental.pallas import tpu as pltpu
from jax.experimental.pallas import tpu_sc as plsc

_VOCAB = 100000
_DIM = 64
_CDIM = 16
_B = 1024
_L = 200
_LT = 20

_NC = 2
_NS = 16
_NW = _NC * _NS

_TXT = _B * _L
_TOP = _B * _LT
_WRD = _TXT + _TOP
_CHUNK = 128
_WRD_PW = _WRD // _NW
_POS_PW = _TXT // _NW
_WRD_CH = _WRD_PW // _CHUNK
_POS_CH = _POS_PW // _CHUNK


def _sc_body(word_hbm, c0_hbm, c1_hbm, c2_hbm, c3_hbm,
             wrd_hbm, i0_hbm, i1_hbm, i2_hbm, i3_hbm,
             wrd_out, conn_out, pos_out,
             idx_wrd, idx_pos,
             wb0, wb1, wb2, wb3, p0, p1, p2, p3, cb0, cb1, cb2, cb3,
             wg0, wg1, wg2, wg3, ws0, ws1, ws2, ws3,
             pg0, pg1, pg2, pg3, pa0, pa1, pa2, pa3,
             pso0, pso1, pso2, pso3):
  w = lax.axis_index("s") * _NC + lax.axis_index("c")
  wrd_base = w * _WRD_PW
  pos_base = w * _POS_PW

  wb = [wb0, wb1, wb2, wb3]
  pb = [p0, p1, p2, p3]
  cb = [cb0, cb1, cb2, cb3]
  wg = [wg0, wg1, wg2, wg3]
  ws = [ws0, ws1, ws2, ws3]
  pg = [pg0, pg1, pg2, pg3]
  pa = [pa0, pa1, pa2, pa3]
  pso = [pso0, pso1, pso2, pso3]

  pltpu.sync_copy(wrd_hbm.at[w], idx_wrd)
  pltpu.sync_copy(i0_hbm.at[w], idx_pos.at[0])
  pltpu.sync_copy(i1_hbm.at[w], idx_pos.at[1])
  pltpu.sync_copy(i2_hbm.at[w], idx_pos.at[2])
  pltpu.sync_copy(i3_hbm.at[w], idx_pos.at[3])

  def j1_fire(j, s):
    pltpu.async_copy(word_hbm.at[idx_wrd.at[j]], wb[s], wg[s])

  def j1_a(j, s):
    pltpu.make_async_copy(word_hbm.at[idx_wrd.at[j]], wb[s], wg[s]).wait()
    pltpu.async_copy(
        wb[s], wrd_out.at[pl.ds(wrd_base + j * _CHUNK, _CHUNK)], ws[s])

  def j1_c(j, s):
    pltpu.make_async_copy(
        wb[s], wrd_out.at[pl.ds(wrd_base + j * _CHUNK, _CHUNK)], ws[s]).wait()

  def j2_fire(j, s):
    pltpu.async_copy(word_hbm.at[idx_pos.at[0, j]], pb[s], pg[s])
    pltpu.async_copy(c0_hbm.at[idx_pos.at[0, j]], cb[s], pg[s])

  def j2_a(j, s):
    pltpu.make_async_copy(word_hbm.at[idx_pos.at[0, j]], pb[s], pg[s]).wait()
    pltpu.make_async_copy(c0_hbm.at[idx_pos.at[0, j]], cb[s], pg[s]).wait()
    pltpu.async_copy(word_hbm.at[idx_pos.at[1, j]], pb[s], pa[s], add=True)
    pltpu.async_copy(word_hbm.at[idx_pos.at[2, j]], pb[s], pa[s], add=True)
    pltpu.async_copy(word_hbm.at[idx_pos.at[3, j]], pb[s], pa[s], add=True)
    pltpu.async_copy(c1_hbm.at[idx_pos.at[1, j]], cb[s], pa[s], add=True)
    pltpu.async_copy(c2_hbm.at[idx_pos.at[2, j]], cb[s], pa[s], add=True)
    pltpu.async_copy(c3_hbm.at[idx_pos.at[3, j]], cb[s], pa[s], add=True)

  def j2_b(j, s):
    dst = pl.ds(pos_base + j * _CHUNK, _CHUNK)
    pltpu.make_async_copy(word_hbm.at[idx_pos.at[1, j]], pb[s], pa[s]).wait()
    pltpu.make_async_copy(word_hbm.at[idx_pos.at[2, j]], pb[s], pa[s]).wait()
    pltpu.make_async_copy(word_hbm.at[idx_pos.at[3, j]], pb[s], pa[s]).wait()
    pltpu.make_async_copy(c1_hbm.at[idx_pos.at[1, j]], cb[s], pa[s]).wait()
    pltpu.make_async_copy(c2_hbm.at[idx_pos.at[2, j]], cb[s], pa[s]).wait()
    pltpu.make_async_copy(c3_hbm.at[idx_pos.at[3, j]], cb[s], pa[s]).wait()
    pltpu.async_copy(pb[s], pos_out.at[dst], pso[s])
    pltpu.async_copy(cb[s], conn_out.at[dst], pso[s])

  def j2_c(j, s):
    dst = pl.ds(pos_base + j * _CHUNK, _CHUNK)
    pltpu.make_async_copy(pb[s], pos_out.at[dst], pso[s]).wait()
    pltpu.make_async_copy(cb[s], conn_out.at[dst], pso[s]).wait()

  j2_fire(0, 0)
  j2_fire(1, 1)
  j1_fire(0, 0)
  j1_fire(1, 1)
  j1_a(0, 0)
  j1_fire(2, 2)
  j1_a(1, 1)
  j1_fire(3, 3)
  j1_a(2, 2)
  j1_c(0, 0)
  j1_fire(4, 0)
  j1_a(3, 3)
  j1_c(1, 1)
  j1_fire(5, 1)
  j1_a(4, 0)
  j1_c(2, 2)
  j1_fire(6, 2)
  j2_a(0, 0)
  j2_fire(2, 2)
  j2_a(1, 1)
  j2_b(0, 0)
  j2_fire(3, 3)

  def main_loop(t, _):
    for u in range(4):
      j = 4 * t + 2 + u
      k = j + 3
      j2_a(j, (2 + u) % 4)
      j1_a(k, (1 + u) % 4)
      j2_b(j - 1, (1 + u) % 4)
      j1_c(k - 2, (3 + u) % 4)
      j1_fire(k + 2, (3 + u) % 4)
      j2_c(j - 2, u)
      @pl.when(j + 2 < _POS_CH)
      def _():
        j2_fire(j + 2, u)
    return _

  lax.fori_loop(0, (_POS_CH - 2) // 4, main_loop, None)

  j1_a(_WRD_CH - 2, (_WRD_CH - 2) % 4)
  j1_a(_WRD_CH - 1, (_WRD_CH - 1) % 4)
  j2_b(_POS_CH - 1, (_POS_CH - 1) % 4)
  j1_c(_WRD_CH - 4, (_WRD_CH - 4) % 4)
  j1_c(_WRD_CH - 3, (_WRD_CH - 3) % 4)
  j1_c(_WRD_CH - 2, (_WRD_CH - 2) % 4)
  j1_c(_WRD_CH - 1, (_WRD_CH - 1) % 4)
  j2_c(_POS_CH - 2, (_POS_CH - 2) % 4)
  j2_c(_POS_CH - 1, (_POS_CH - 1) % 4)


@jax.jit
def _run(word_table, c0, c1, c2, c3, wrd_idx, i0, i1, i2, i3):
  mesh = plsc.VectorSubcoreMesh(core_axis_name="c", subcore_axis_name="s")
  f = pl.kernel(
      _sc_body,
      out_type=[
          jax.ShapeDtypeStruct((_WRD, _DIM), jnp.float32),
          jax.ShapeDtypeStruct((_TXT, _CDIM), jnp.float32),
          jax.ShapeDtypeStruct((_TXT, _DIM), jnp.float32),
      ],
      mesh=mesh,
      compiler_params=pltpu.CompilerParams(use_tc_tiling_on_sc=False),
      scratch_types=[
          pltpu.VMEM((_WRD_CH, _CHUNK), jnp.int32),
          pltpu.VMEM((4, _POS_CH, _CHUNK), jnp.int32),
      ] + [pltpu.VMEM((_CHUNK, _DIM), jnp.float32)] * 4
        + [pltpu.VMEM((_CHUNK, _DIM), jnp.float32)] * 4
        + [pltpu.VMEM((_CHUNK, _CDIM), jnp.float32)] * 4
        + [pltpu.SemaphoreType.DMA] * 20,
  )
  return f(word_table, c0, c1, c2, c3, wrd_idx, i0, i1, i2, i3)


def kernel(word_table, conn_NOUN, conn_VERB, conn_ADJ, conn_ADV,
           text, topic, idx_NOUN, idx_VERB, idx_ADJ, idx_ADV,
           txt_l, top_l):
  wrd_idx = jnp.concatenate([text.reshape(-1), topic.reshape(-1)])
  wrd_idx = wrd_idx.reshape(_NW, _WRD_CH, _CHUNK)
  i0 = idx_NOUN.reshape(_NW, _POS_CH, _CHUNK)
  i1 = idx_VERB.reshape(_NW, _POS_CH, _CHUNK)
  i2 = idx_ADJ.reshape(_NW, _POS_CH, _CHUNK)
  i3 = idx_ADV.reshape(_NW, _POS_CH, _CHUNK)
  wrd_E, conn_E, pos_E = _run(
      word_table, conn_NOUN, conn_VERB, conn_ADJ, conn_ADV,
      wrd_idx, i0, i1, i2, i3)
  word_E = wrd_E[:_TXT].reshape(_B, _L, _DIM)
  top_E = wrd_E[_TXT:].reshape(_B, _LT, _DIM)
  return (word_E, top_E,
          conn_E.reshape(_B, _L, _CDIM),
          pos_E.reshape(_B, _L, _DIM),
          txt_l, top_l)


# speedup vs baseline: 1.5078x; 1.0009x over previous
import jax
import jax.numpy as jnp
from jax import lax
from jax.experimental import pallas as pl
from jax.experimental.pallas import tpu as pltpu
from jax.experimental.pallas import tpu_sc as plsc

_VOCAB = 100000
_DIM = 64
_CDIM = 16
_B = 1024
_L = 200
_LT = 20

_NC = 2
_NS = 16
_NW = _NC * _NS

_TXT = _B * _L
_TOP = _B * _LT
_WRD = _TXT + _TOP
_CHUNK = 128
_WRD_PW = _WRD // _NW
_POS_PW = _TXT // _NW
_WRD_CH = _WRD_PW // _CHUNK
_POS_CH = _POS_PW // _CHUNK


def _sc_body(wj_hbm, w0_hbm, w1_hbm, w2_hbm, w3_hbm,
             c0_hbm, c1_hbm, c2_hbm, c3_hbm,
             wrd_hbm, i0_hbm, i1_hbm, i2_hbm, i3_hbm,
             wrd_out, conn_out, pos_out,
             idx_wrd, idx_pos,
             wb0, wb1, wb2, wb3, p0, p1, p2, p3, cb0, cb1, cb2, cb3,
             wg0, wg1, wg2, wg3, ws0, ws1, ws2, ws3,
             pg0, pg1, pg2, pg3, pa0, pa1, pa2, pa3,
             pso0, pso1, pso2, pso3):
  w = lax.axis_index("s") * _NC + lax.axis_index("c")
  wrd_base = w * _WRD_PW
  pos_base = w * _POS_PW

  wb = [wb0, wb1, wb2, wb3]
  pb = [p0, p1, p2, p3]
  cb = [cb0, cb1, cb2, cb3]
  wg = [wg0, wg1, wg2, wg3]
  ws = [ws0, ws1, ws2, ws3]
  pg = [pg0, pg1, pg2, pg3]
  pa = [pa0, pa1, pa2, pa3]
  pso = [pso0, pso1, pso2, pso3]

  pltpu.sync_copy(wrd_hbm.at[w], idx_wrd)
  pltpu.sync_copy(i0_hbm.at[w], idx_pos.at[0])
  pltpu.sync_copy(i1_hbm.at[w], idx_pos.at[1])
  pltpu.sync_copy(i2_hbm.at[w], idx_pos.at[2])
  pltpu.sync_copy(i3_hbm.at[w], idx_pos.at[3])

  def j1_fire(j, s):
    pltpu.async_copy(wj_hbm.at[idx_wrd.at[j]], wb[s], wg[s])

  def j1_a(j, s):
    pltpu.make_async_copy(wj_hbm.at[idx_wrd.at[j]], wb[s], wg[s]).wait()
    pltpu.async_copy(
        wb[s], wrd_out.at[pl.ds(wrd_base + j * _CHUNK, _CHUNK)], ws[s])

  def j1_c(j, s):
    pltpu.make_async_copy(
        wb[s], wrd_out.at[pl.ds(wrd_base + j * _CHUNK, _CHUNK)], ws[s]).wait()

  def j2_fire(j, s):
    pltpu.async_copy(w0_hbm.at[idx_pos.at[0, j]], pb[s], pg[s])
    pltpu.async_copy(c0_hbm.at[idx_pos.at[0, j]], cb[s], pg[s])

  def j2_a(j, s):
    pltpu.make_async_copy(w0_hbm.at[idx_pos.at[0, j]], pb[s], pg[s]).wait()
    pltpu.make_async_copy(c0_hbm.at[idx_pos.at[0, j]], cb[s], pg[s]).wait()
    pltpu.async_copy(w1_hbm.at[idx_pos.at[1, j]], pb[s], pa[s], add=True)
    pltpu.async_copy(w2_hbm.at[idx_pos.at[2, j]], pb[s], pa[s], add=True)
    pltpu.async_copy(w3_hbm.at[idx_pos.at[3, j]], pb[s], pa[s], add=True)
    pltpu.async_copy(c1_hbm.at[idx_pos.at[1, j]], cb[s], pa[s], add=True)
    pltpu.async_copy(c2_hbm.at[idx_pos.at[2, j]], cb[s], pa[s], add=True)
    pltpu.async_copy(c3_hbm.at[idx_pos.at[3, j]], cb[s], pa[s], add=True)

  def j2_b(j, s):
    dst = pl.ds(pos_base + j * _CHUNK, _CHUNK)
    pltpu.make_async_copy(w1_hbm.at[idx_pos.at[1, j]], pb[s], pa[s]).wait()
    pltpu.make_async_copy(w2_hbm.at[idx_pos.at[2, j]], pb[s], pa[s]).wait()
    pltpu.make_async_copy(w3_hbm.at[idx_pos.at[3, j]], pb[s], pa[s]).wait()
    pltpu.make_async_copy(c1_hbm.at[idx_pos.at[1, j]], cb[s], pa[s]).wait()
    pltpu.make_async_copy(c2_hbm.at[idx_pos.at[2, j]], cb[s], pa[s]).wait()
    pltpu.make_async_copy(c3_hbm.at[idx_pos.at[3, j]], cb[s], pa[s]).wait()
    pltpu.async_copy(pb[s], pos_out.at[dst], pso[s])
    pltpu.async_copy(cb[s], conn_out.at[dst], pso[s])

  def j2_c(j, s):
    dst = pl.ds(pos_base + j * _CHUNK, _CHUNK)
    pltpu.make_async_copy(pb[s], pos_out.at[dst], pso[s]).wait()
    pltpu.make_async_copy(cb[s], conn_out.at[dst], pso[s]).wait()

  j2_fire(0, 0)
  j2_fire(1, 1)
  j1_fire(0, 0)
  j1_fire(1, 1)
  j1_a(0, 0)
  j1_fire(2, 2)
  j1_a(1, 1)
  j1_fire(3, 3)
  j1_a(2, 2)
  j1_c(0, 0)
  j1_fire(4, 0)
  j1_a(3, 3)
  j1_c(1, 1)
  j1_fire(5, 1)
  j1_a(4, 0)
  j1_c(2, 2)
  j1_fire(6, 2)
  j2_a(0, 0)
  j2_fire(2, 2)
  j2_a(1, 1)
  j2_b(0, 0)
  j2_fire(3, 3)

  def main_loop(t, _):
    for u in range(4):
      j = 4 * t + 2 + u
      k = j + 3
      j2_a(j, (2 + u) % 4)
      j1_a(k, (1 + u) % 4)
      j2_b(j - 1, (1 + u) % 4)
      j1_c(k - 2, (3 + u) % 4)
      j1_fire(k + 2, (3 + u) % 4)
      j2_c(j - 2, u)
      @pl.when(j + 2 < _POS_CH)
      def _():
        j2_fire(j + 2, u)
    return _

  lax.fori_loop(0, (_POS_CH - 2) // 4, main_loop, None)

  j1_a(_WRD_CH - 2, (_WRD_CH - 2) % 4)
  j1_a(_WRD_CH - 1, (_WRD_CH - 1) % 4)
  j2_b(_POS_CH - 1, (_POS_CH - 1) % 4)
  j1_c(_WRD_CH - 4, (_WRD_CH - 4) % 4)
  j1_c(_WRD_CH - 3, (_WRD_CH - 3) % 4)
  j1_c(_WRD_CH - 2, (_WRD_CH - 2) % 4)
  j1_c(_WRD_CH - 1, (_WRD_CH - 1) % 4)
  j2_c(_POS_CH - 2, (_POS_CH - 2) % 4)
  j2_c(_POS_CH - 1, (_POS_CH - 1) % 4)


@jax.jit
def _run(word_table, c0, c1, c2, c3, wrd_idx, i0, i1, i2, i3):
  wt = word_table
  mesh = plsc.VectorSubcoreMesh(core_axis_name="c", subcore_axis_name="s")
  f = pl.kernel(
      _sc_body,
      out_type=[
          jax.ShapeDtypeStruct((_WRD, _DIM), jnp.float32),
          jax.ShapeDtypeStruct((_TXT, _CDIM), jnp.float32),
          jax.ShapeDtypeStruct((_TXT, _DIM), jnp.float32),
      ],
      mesh=mesh,
      compiler_params=pltpu.CompilerParams(use_tc_tiling_on_sc=False),
      scratch_types=[
          pltpu.VMEM((_WRD_CH, _CHUNK), jnp.int32),
          pltpu.VMEM((4, _POS_CH, _CHUNK), jnp.int32),
      ] + [pltpu.VMEM((_CHUNK, _DIM), jnp.float32)] * 4
        + [pltpu.VMEM((_CHUNK, _DIM), jnp.float32)] * 4
        + [pltpu.VMEM((_CHUNK, _CDIM), jnp.float32)] * 4
        + [pltpu.SemaphoreType.DMA] * 20,
  )
  return f(wt, wt, wt, wt, wt, c0, c1, c2, c3, wrd_idx, i0, i1, i2, i3)


def kernel(word_table, conn_NOUN, conn_VERB, conn_ADJ, conn_ADV,
           text, topic, idx_NOUN, idx_VERB, idx_ADJ, idx_ADV,
           txt_l, top_l):
  wrd_idx = jnp.concatenate([text.reshape(-1), topic.reshape(-1)])
  wrd_idx = wrd_idx.reshape(_NW, _WRD_CH, _CHUNK)
  i0 = idx_NOUN.reshape(_NW, _POS_CH, _CHUNK)
  i1 = idx_VERB.reshape(_NW, _POS_CH, _CHUNK)
  i2 = idx_ADJ.reshape(_NW, _POS_CH, _CHUNK)
  i3 = idx_ADV.reshape(_NW, _POS_CH, _CHUNK)
  wrd_E, conn_E, pos_E = _run(
      word_table, conn_NOUN, conn_VERB, conn_ADJ, conn_ADV,
      wrd_idx, i0, i1, i2, i3)
  word_E = wrd_E[:_TXT].reshape(_B, _L, _DIM)
  top_E = wrd_E[_TXT:].reshape(_B, _LT, _DIM)
  return (word_E, top_E,
          conn_E.reshape(_B, _L, _CDIM),
          pos_E.reshape(_B, _L, _DIM),
          txt_l, top_l)
